# Initial kernel scaffold; baseline (speedup 1.0000x reference)
import jax
import jax.numpy as jnp
from jax.experimental import pallas as pl


def kernel(x, stable, dim, descending, values, indices):
    raise NotImplementedError("write your pallas kernel here")


# speedup vs baseline: 1.9463x; 1.9463x over previous
import functools

import jax
import jax.numpy as jnp
from jax import lax
from jax.experimental import pallas as pl
from jax.experimental.pallas import tpu as pltpu
from jax.experimental.pallas import tpu_sc as plsc

_ROWS = 64
_N = 8192
_LANES = 16
_VREGS = _N // _LANES
_NC = 2
_NS = 16
_NW = _NC * _NS
_ROWS_PER_W = _ROWS // _NW
_RADIX_BITS = 8
_RADIX = 1 << _RADIX_BITS
_PASSES = 4
_MIN32 = jnp.int32(-0x80000000)


def _sc_sort_rows(xm):
    mesh = plsc.VectorSubcoreMesh(core_axis_name="c", subcore_axis_name="s")

    @functools.partial(
        pl.kernel,
        out_type=[
            jax.ShapeDtypeStruct((_ROWS, _N), jnp.float32),
            jax.ShapeDtypeStruct((_ROWS, _N), jnp.int32),
        ],
        mesh=mesh,
        compiler_params=pltpu.CompilerParams(needs_layout_passes=False),
        scratch_types=[
            pltpu.VMEM((_N,), jnp.float32),
            pltpu.VMEM((_N,), jnp.float32),
            pltpu.VMEM((_N,), jnp.int32),
            pltpu.VMEM((_N,), jnp.int32),
            pltpu.VMEM((_RADIX,), jnp.int32),
        ],
    )
    def sort_kernel(x_hbm, vals_hbm, idx_hbm, ka, kb, ia, ib, hist):
        wid = lax.axis_index("s") * _NC + lax.axis_index("c")
        lane_iota = lax.iota(jnp.int32, _LANES)

        def zero_hist():
            zeros = jnp.zeros((_LANES,), jnp.int32)
            for j in range(_RADIX // _LANES):
                hist[pl.ds(j * _LANES, _LANES)] = zeros

        def hist_to_offsets():
            def off_body(j, running):
                h = hist[pl.ds(j * _LANES, _LANES)]
                inc = plsc.cumsum(h)
                hist[pl.ds(j * _LANES, _LANES)] = inc - h + running
                return running + jnp.sum(h)

            lax.fori_loop(0, _RADIX // _LANES, off_body, jnp.int32(0))

        def sort_one_row(rr, _):
            row = wid * _ROWS_PER_W + rr

            pltpu.sync_copy(x_hbm.at[row], ka)

            zero_hist()

            def pro_body(i, _):
                sl = pl.ds(i * _LANES, _LANES)
                b = plsc.bitcast(ka[sl], jnp.int32)
                u = b ^ ((b >> 31) | _MIN32)
                ka[sl] = plsc.bitcast(u, jnp.float32)
                ia[sl] = i * _LANES + lane_iota
                d = u & (_RADIX - 1)
                cnt, last_m = plsc.scan_count(d)
                plsc.addupdate_scatter(hist, [d], cnt, mask=last_m)
                return 0

            lax.fori_loop(0, _VREGS, pro_body, 0)

            bufs = [(ka, ia), (kb, ib)]
            for p in range(_PASSES):
                k_in, i_in = bufs[p % 2]
                k_out, i_out = bufs[(p + 1) % 2]
                shift = p * _RADIX_BITS

                if p > 0:
                    zero_hist()

                    def hist_body(i, _, k_in=k_in, shift=shift):
                        sl = pl.ds(i * _LANES, _LANES)
                        u = plsc.bitcast(k_in[sl], jnp.int32)
                        d = lax.shift_right_logical(u, shift) & (_RADIX - 1)
                        cnt, last_m = plsc.scan_count(d)
                        plsc.addupdate_scatter(hist, [d], cnt, mask=last_m)
                        return 0

                    lax.fori_loop(0, _VREGS, hist_body, 0)

                hist_to_offsets()

                last_pass = p == _PASSES - 1

                def perm_body(i, _, k_in=k_in, i_in=i_in, k_out=k_out,
                              i_out=i_out, shift=shift, last_pass=last_pass):
                    sl = pl.ds(i * _LANES, _LANES)
                    u = plsc.bitcast(k_in[sl], jnp.int32)
                    ix = i_in[sl]
                    d = lax.shift_right_logical(u, shift) & (_RADIX - 1)
                    cnt, last_m = plsc.scan_count(d)
                    base = plsc.load_gather(hist, [d])
                    dest = base + cnt - 1
                    if last_pass:
                        out_bits = u ^ (jnp.invert(u >> 31) | _MIN32)
                        store = plsc.bitcast(out_bits, jnp.float32)
                    else:
                        store = plsc.bitcast(u, jnp.float32)
                    plsc.store_scatter(k_out, [dest], store)
                    plsc.store_scatter(i_out, [dest], ix)
                    plsc.addupdate_scatter(hist, [d], cnt, mask=last_m)
                    return 0

                lax.fori_loop(0, _VREGS, perm_body, 0)

            pltpu.sync_copy(ka, vals_hbm.at[row])
            pltpu.sync_copy(ia, idx_hbm.at[row])
            return 0

        lax.fori_loop(0, _ROWS_PER_W, sort_one_row, 0)

    return sort_kernel(xm)


def kernel(x, stable, dim, descending, values, indices):
    del stable, dim, values, indices
    desc = jnp.asarray(descending)
    xm = jnp.where(desc, -x, x)
    vals_m, idx = _sc_sort_rows(xm)
    vals = jnp.where(desc, -vals_m, vals_m)
    return vals, idx.astype(jnp.int64)


# speedup vs baseline: 2.7309x; 1.4031x over previous
import functools

import jax
import jax.numpy as jnp
from jax import lax
from jax.experimental import pallas as pl
from jax.experimental.pallas import tpu as pltpu
from jax.experimental.pallas import tpu_sc as plsc

_ROWS = 64
_N = 8192
_LANES = 16
_VREGS = _N // _LANES
_NC = 2
_NS = 16
_NW = _NC * _NS
_ROWS_PER_W = _ROWS // _NW
_RADIX_BITS = 8
_RADIX = 1 << _RADIX_BITS
_PASSES = 4
_MIN32 = jnp.int32(-0x80000000)


def _sc_sort_rows(xm):
    mesh = plsc.VectorSubcoreMesh(core_axis_name="c", subcore_axis_name="s")

    @functools.partial(
        pl.kernel,
        out_type=[
            jax.ShapeDtypeStruct((_ROWS, _N), jnp.float32),
            jax.ShapeDtypeStruct((_ROWS, _N), jnp.int32),
        ],
        mesh=mesh,
        compiler_params=pltpu.CompilerParams(needs_layout_passes=False),
        scratch_types=[
            pltpu.VMEM((_N,), jnp.float32),
            pltpu.VMEM((_N,), jnp.float32),
            pltpu.VMEM((_N,), jnp.int32),
            pltpu.VMEM((_N,), jnp.int32),
            pltpu.VMEM((_RADIX,), jnp.int32),
            pltpu.VMEM((_RADIX,), jnp.int32),
        ],
    )
    def sort_kernel(x_hbm, vals_hbm, idx_hbm, ka, kb, ia, ib, hist0, hist1):
        wid = lax.axis_index("s") * _NC + lax.axis_index("c")
        lane_iota = lax.iota(jnp.int32, _LANES)

        def zero_hist(hist):
            zeros = jnp.zeros((_LANES,), jnp.int32)
            for j in range(_RADIX // _LANES):
                hist[pl.ds(j * _LANES, _LANES)] = zeros

        def hist_to_offsets(hist):
            def off_body(j, running):
                h = hist[pl.ds(j * _LANES, _LANES)]
                inc = plsc.cumsum(h)
                hist[pl.ds(j * _LANES, _LANES)] = inc - h + running
                return running + jnp.sum(h)

            lax.fori_loop(0, _RADIX // _LANES, off_body, jnp.int32(0))

        def sort_one_row(rr, _):
            row = wid * _ROWS_PER_W + rr

            pltpu.sync_copy(x_hbm.at[row], ka)

            zero_hist(hist0)

            def pro_body(i, _):
                sl = pl.ds(i * _LANES, _LANES)
                b = plsc.bitcast(ka[sl], jnp.int32)
                u = b ^ ((b >> 31) | _MIN32)
                ka[sl] = plsc.bitcast(u, jnp.float32)
                ia[sl] = i * _LANES + lane_iota
                d = u & (_RADIX - 1)
                cnt, last_m = plsc.scan_count(d)
                plsc.addupdate_scatter(hist0, [d], cnt, mask=last_m)
                return 0

            lax.fori_loop(0, _VREGS, pro_body, 0)

            bufs = [(ka, ia), (kb, ib)]
            hists = [hist0, hist1]
            for p in range(_PASSES):
                k_in, i_in = bufs[p % 2]
                k_out, i_out = bufs[(p + 1) % 2]
                hist = hists[p % 2]
                hist_nxt = hists[(p + 1) % 2]
                shift = p * _RADIX_BITS

                hist_to_offsets(hist)

                last_pass = p == _PASSES - 1
                if not last_pass:
                    zero_hist(hist_nxt)

                def perm_body(i, _, k_in=k_in, i_in=i_in, k_out=k_out,
                              i_out=i_out, hist=hist, hist_nxt=hist_nxt,
                              shift=shift, last_pass=last_pass):
                    sl = pl.ds(i * _LANES, _LANES)
                    u = plsc.bitcast(k_in[sl], jnp.int32)
                    ix = i_in[sl]
                    d = lax.shift_right_logical(u, shift) & (_RADIX - 1)
                    cnt, last_m = plsc.scan_count(d)
                    base = plsc.load_gather(hist, [d])
                    dest = base + cnt - 1
                    if last_pass:
                        out_bits = u ^ (jnp.invert(u >> 31) | _MIN32)
                        store = plsc.bitcast(out_bits, jnp.float32)
                    else:
                        store = plsc.bitcast(u, jnp.float32)
                    plsc.store_scatter(k_out, [dest], store)
                    plsc.store_scatter(i_out, [dest], ix)
                    plsc.addupdate_scatter(hist, [d], cnt, mask=last_m)
                    if not last_pass:
                        d2 = lax.shift_right_logical(
                            u, shift + _RADIX_BITS) & (_RADIX - 1)
                        cnt2, last2 = plsc.scan_count(d2)
                        plsc.addupdate_scatter(hist_nxt, [d2], cnt2, mask=last2)
                    return 0

                lax.fori_loop(0, _VREGS, perm_body, 0)

            pltpu.sync_copy(ka, vals_hbm.at[row])
            pltpu.sync_copy(ia, idx_hbm.at[row])
            return 0

        lax.fori_loop(0, _ROWS_PER_W, sort_one_row, 0)

    return sort_kernel(xm)


def kernel(x, stable, dim, descending, values, indices):
    del stable, dim, values, indices
    desc = jnp.asarray(descending)
    xm = jnp.where(desc, -x, x)
    vals_m, idx = _sc_sort_rows(xm)
    vals = jnp.where(desc, -vals_m, vals_m)
    return vals, idx.astype(jnp.int64)
